# Initial kernel scaffold; baseline (speedup 1.0000x reference)
import jax
import jax.numpy as jnp
from jax.experimental import pallas as pl


def kernel(x, edge_index, edge_attr, batch, params):
    raise NotImplementedError("write your pallas kernel here")


# speedup vs baseline: 1.1800x; 1.1800x over previous
import functools

import jax
import jax.numpy as jnp
from jax import lax
from jax.experimental import pallas as pl
from jax.experimental.pallas import tpu as pltpu
from jax.experimental.pallas import tpu_sc as plsc

F32 = jnp.float32
I32 = jnp.int32

_NC = 2
_NS = 16
_NT = _NC * _NS
_C = 128


def _mesh():
    return plsc.VectorSubcoreMesh(
        core_axis_name="c", subcore_axis_name="s", num_cores=_NC,
        num_subcores=_NS)


def _rup(x, m):
    return (x + m - 1) // m * m


@functools.lru_cache(maxsize=None)
def _make_sc_fea(nn, ep, p, remap):
    ept = ep // _NT
    nchunks = ept // _C
    pp = p * p
    zn = 2000

    def body(*refs):
        (src_h, dst_h, a1_h, a2_h, a3_h, np_h) = refs[:6]
        nout = 4 if remap else 1
        outs = refs[6:6 + nout]
        fea_h = outs[0]
        (zv, srcc, dstc, a1c, a2c, a3c, alc, fic, nsc, ndc, kic,
         feas, sem, sem2) = refs[6 + nout:]
        cid = lax.axis_index("c")
        sid = lax.axis_index("s")
        wid = sid * _NC + cid

        def zz(i, _):
            zv[pl.ds(16 * i, 16)] = jnp.zeros((16,), F32)
            return _
        lax.fori_loop(0, zn // 16, zz, None)

        @pl.when(sid == 0)
        def _():
            for q in range(pp // zn):
                pltpu.sync_copy(zv, feas.at[pl.ds(q * zn, zn)])

        plsc.subcore_barrier()

        ebase = wid * ept

        def chunk(ci, _):
            off = ebase + ci * _C
            pltpu.sync_copy(src_h.at[pl.ds(off, _C)], srcc)
            pltpu.sync_copy(dst_h.at[pl.ds(off, _C)], dstc)
            pltpu.sync_copy(a1_h.at[pl.ds(off, _C)], a1c)
            pltpu.sync_copy(a2_h.at[pl.ds(off, _C)], a2c)
            pltpu.sync_copy(a3_h.at[pl.ds(off, _C)], a3c)
            if remap:
                g1 = pltpu.async_copy(np_h.at[srcc], nsc, sem2)
                g2 = pltpu.async_copy(np_h.at[dstc], ndc, sem2)
            for j in range(_C // 16):
                sl = pl.ds(16 * j, 16)
                sv = srcc[sl]
                dv = dstc[sl]
                fic[sl] = (sv % p) * p + (dv % p)
                alc[sl] = a1c[sl] + a2c[sl] + a3c[sl]
            if remap:
                g1.wait()
                g2.wait()
                for j in range(_C // 16):
                    sl = pl.ds(16 * j, 16)
                    ns = nsc[sl]
                    nd = ndc[sl]
                    keep = (ns >= 0) & (nd >= 0)
                    nsc[sl] = jnp.where(keep, ns, 0)
                    ndc[sl] = jnp.where(keep, nd, 0)
                    kic[sl] = jnp.where(keep, 1, 0)
            pltpu.async_copy(alc, feas.at[fic], sem, add=True).wait()
            if remap:
                pltpu.sync_copy(nsc, outs[1].at[pl.ds(off, _C)])
                pltpu.sync_copy(ndc, outs[2].at[pl.ds(off, _C)])
                pltpu.sync_copy(kic, outs[3].at[pl.ds(off, _C)])
            return _
        lax.fori_loop(0, nchunks, chunk, None)

        plsc.subcore_barrier()

        @pl.when(sid == 0)
        def _():
            pltpu.sync_copy(feas, fea_h.at[cid])

    out_type = [jax.ShapeDtypeStruct((_NC, pp), F32)]
    if remap:
        out_type += [jax.ShapeDtypeStruct((ep,), I32),
                     jax.ShapeDtypeStruct((ep,), I32),
                     jax.ShapeDtypeStruct((ep,), I32)]
    return pl.kernel(
        body,
        out_type=out_type,
        mesh=_mesh(),
        scratch_types=[
            pltpu.VMEM((zn,), F32),
            pltpu.VMEM((_C,), I32),
            pltpu.VMEM((_C,), I32),
            pltpu.VMEM((_C,), F32),
            pltpu.VMEM((_C,), F32),
            pltpu.VMEM((_C,), F32),
            pltpu.VMEM((_C,), F32),
            pltpu.VMEM((_C,), I32),
            pltpu.VMEM((_C,), I32),
            pltpu.VMEM((_C,), I32),
            pltpu.VMEM((_C,), I32),
            pltpu.VMEM_SHARED((pp,), F32),
            pltpu.SemaphoreType.DMA,
            pltpu.SemaphoreType.DMA,
        ],
    )


def _sc_fea(src, dst, alphas, newpos, p, remap):
    nn = newpos.shape[0]
    ep = src.shape[0]
    return _make_sc_fea(nn, ep, p, remap)(
        src, dst, alphas[0], alphas[1], alphas[2], newpos)


def _tc_topk(score_g, k):
    g, nper = score_g.shape

    def f(s_ref, perm_ref, np_ref):
        gi = pl.program_id(0)
        srow = s_ref[...].reshape(1, nper)
        ii = lax.broadcasted_iota(I32, (nper, nper), 0)
        ji = lax.broadcasted_iota(I32, (nper, nper), 1)
        ident = (ii == ji).astype(F32)
        scol = lax.dot_general(ident, srow, (((1,), (1,)), ((), ())),
                               precision=lax.Precision.HIGHEST,
                               preferred_element_type=F32)
        gt = (srow > scol).astype(F32)
        eq = (srow == scol).astype(F32)
        jlt = (ji < ii).astype(F32)
        rank = jnp.sum(gt + eq * jlt, axis=1, keepdims=True)
        keep = rank < float(k)
        npos = jnp.where(keep, float(k) * gi + rank, -1.0)
        np_ref[...] = lax.dot_general(
            npos, ident, (((0,), (0,)), ((), ())),
            precision=lax.Precision.HIGHEST,
            preferred_element_type=F32).astype(I32).reshape(1, 1, nper)
        riota = lax.broadcasted_iota(I32, (nper, k), 1).astype(F32)
        m = ((jnp.broadcast_to(rank, (nper, k)) == riota) &
             jnp.broadcast_to(keep, (nper, k))).astype(F32)
        irow = lax.broadcasted_iota(I32, (1, nper), 1).astype(F32)
        perm = lax.dot_general(irow, m, (((1,), (0,)), ((), ())),
                               precision=lax.Precision.HIGHEST,
                               preferred_element_type=F32)
        perm_ref[...] = (perm + float(nper) * gi).astype(I32).reshape(1, 1, k)

    return pl.pallas_call(
        f,
        grid=(g,),
        in_specs=[pl.BlockSpec((1, 1, nper), lambda i: (i, 0, 0))],
        out_specs=[pl.BlockSpec((1, 1, k), lambda i: (i, 0, 0)),
                   pl.BlockSpec((1, 1, nper), lambda i: (i, 0, 0))],
        out_shape=[jax.ShapeDtypeStruct((g, 1, k), I32),
                   jax.ShapeDtypeStruct((g, 1, nper), I32)],
    )(score_g.reshape(g, 1, nper))


def _bn(x, g, b):
    mu = x.mean(0)
    v = x.var(0)
    return g * (x - mu) * lax.rsqrt(v + 1e-5) + b


def _gat(x, ei, mask, p, n, edge_attr=None):
    loop = jnp.arange(n, dtype=ei.dtype)
    src = jnp.concatenate([ei[0], loop])
    dst = jnp.concatenate([ei[1], loop])
    m = jnp.concatenate([mask, jnp.ones((n,), bool)])
    h = x @ p["W"]
    a = (h * p["att_src"]).sum(-1)[src] + (h * p["att_dst"]).sum(-1)[dst]
    if edge_attr is not None:
        la = jnp.broadcast_to(edge_attr.mean(0, keepdims=True),
                              (n, edge_attr.shape[1]))
        ea = jnp.concatenate([edge_attr, la], 0)
        a = a + ((ea @ p["lin_edge"]) * p["att_edge"]).sum(-1)
    a = jax.nn.leaky_relu(a, 0.2)
    a = jnp.where(m, a, -1e9)
    amax = lax.stop_gradient(jax.ops.segment_max(a, dst, num_segments=n))
    ex = jnp.exp(a - amax[dst]) * m
    den = jax.ops.segment_sum(ex, dst, num_segments=n)
    alpha = ex / (den[dst] + 1e-16)
    out = jax.ops.segment_sum(alpha[:, None] * h[src], dst,
                              num_segments=n) + p["bias"]
    return out, alpha


def _gcb(x, ei, mask, p, n, edge_attr=None):
    x1, a1 = _gat(x, ei, mask, p["g1"], n, edge_attr)
    x1 = jax.nn.relu(x1)
    x1 = _bn(x1, p["bn1_g"], p["bn1_b"])
    x2, a2 = _gat(x1, ei, mask, p["g2"], n, edge_attr)
    if edge_attr is None:
        x2 = jax.nn.relu(x2)
    x2 = _bn(x2, p["bn2_g"], p["bn2_b"])
    x3, a3 = _gat(x2, ei, mask, p["g3"], n, edge_attr)
    x3 = jax.nn.relu(x3)
    x3 = _bn(x3, p["bn3_g"], p["bn3_b"])
    return jnp.concatenate([x1, x2, x3], -1), a1, a2, a3


def _gnorm(x, batch, g, b, ms, ng):
    cnt = jax.ops.segment_sum(jnp.ones((x.shape[0],), F32), batch,
                              num_segments=ng)[:, None]
    mean = jax.ops.segment_sum(x, batch, num_segments=ng) / cnt
    out = x - mean[batch] * ms
    var = jax.ops.segment_sum(out * out, batch, num_segments=ng) / cnt
    return g * out * lax.rsqrt(var[batch] + 1e-5) + b


def _pad1(a, ln, val=0):
    return jnp.pad(a, (0, ln - a.shape[0]), constant_values=val)


def kernel(x, edge_index, edge_attr, batch, params):
    n = x.shape[0]
    e = edge_index.shape[1]
    ng = 50
    nper = n // ng
    k1 = nper // 2
    n_b2 = ng * k1
    k2 = k1 // 2
    ep1 = _rup(e + n, _NT * _C)
    ep2 = _rup(e + n_b2, _NT * _C)

    mask0 = jnp.ones((e,), bool)
    xg, a11, a12, a13 = _gcb(x, edge_index, mask0, params["gc1"], n,
                             edge_attr)

    w1 = params["pool1_w"]
    score = jnp.tanh(xg @ w1 / (jnp.linalg.norm(w1) + 1e-16))
    perm_g, npos_g = _tc_topk(score.reshape(ng, nper), k1)
    perm1 = perm_g.reshape(-1)
    newpos = npos_g.reshape(-1)
    score1 = score[perm1]
    xp = xg[perm1] * score1[:, None]

    loop1 = jnp.arange(n, dtype=I32)
    src1 = _pad1(jnp.concatenate([edge_index[0], loop1]), ep1)
    dst1 = _pad1(jnp.concatenate([edge_index[1], loop1]), ep1)
    al1 = [_pad1(a, ep1) for a in (a11, a12, a13)]
    fea1p, nsrc, ndst, keepi = _sc_fea(src1, dst1, al1, newpos, nper, True)
    fea1 = (fea1p[0] + fea1p[1]).reshape(nper, nper) / (3.0 * ng)

    ei1 = jnp.stack([nsrc[:e], ndst[:e]])
    mask1 = keepi[:e] == 1

    batch1 = jnp.repeat(jnp.arange(ng, dtype=I32), k1)
    xn = _gnorm(xp, batch1, params["gn1_g"], params["gn1_b"],
                params["gn1_ms"], ng)
    xg2, a21, a22, a23 = _gcb(xn, ei1, mask1, params["gc2"], n_b2, None)

    w2 = params["pool2_w"]
    score2f = jnp.tanh(xg2 @ w2 / (jnp.linalg.norm(w2) + 1e-16))
    perm2g, _ = _tc_topk(score2f.reshape(ng, k1), k2)
    perm2 = perm2g.reshape(-1)
    score2 = score2f[perm2]
    xp2 = xg2[perm2] * score2[:, None]

    batch2 = jnp.repeat(jnp.arange(ng, dtype=I32), k2)
    xn2 = _gnorm(xp2, batch2, params["gn2_g"], params["gn2_b"],
                 params["gn2_ms"], ng)
    cnt = jax.ops.segment_sum(jnp.ones((xn2.shape[0],), F32), batch2,
                              num_segments=ng)[:, None]
    pooled = jax.ops.segment_sum(xn2, batch2, num_segments=ng) / cnt

    loop2 = jnp.arange(n_b2, dtype=I32)
    src2 = _pad1(jnp.concatenate([ei1[0], loop2]), ep2)
    dst2 = _pad1(jnp.concatenate([ei1[1], loop2]), ep2)
    al2 = [_pad1(a, ep2) for a in (a21, a22, a23)]
    fea2p = _sc_fea(src2, dst2, al2, jnp.zeros((16,), I32), k1, False)[0]
    fea2 = (fea2p[0] + fea2p[1]).reshape(k1, k1) / (3.0 * ng)

    h = pooled @ params["mlp_W1"] + params["mlp_b1"]
    h = jax.nn.relu(h)
    h = _bn(h, params["mlp_bn1_g"], params["mlp_bn1_b"])
    h = h @ params["mlp_W2"] + params["mlp_b2"]
    h = jax.nn.relu(h)
    h = _bn(h, params["mlp_bn2_g"], params["mlp_bn2_b"])

    return (h, perm1, perm2, score1, score2, batch1, batch2, fea1, fea2)
